# baseline (speedup 1.0000x reference)
import numpy as np
import jax
import jax.numpy as jnp
from jax.experimental import pallas as pl

_RATIOS = np.array([0.5, 1.0, 2.0])
_SCALES = np.array([1.0, 2.0 ** (1.0 / 3.0), 2.0 ** (2.0 / 3.0)])
_LEVELS = [(64, 64, 8, 32), (32, 32, 16, 64), (16, 16, 32, 128), (8, 8, 64, 256)]
_N_ROWS = sum(h * w * 9 for h, w, _, _ in _LEVELS)
_N_PAD = 49152
_ROW_OFF = [0, 36864, 46080, 48384]


def _box_sizes(box_size):
    anchors = box_size * np.tile(_SCALES, (2, len(_RATIOS))).T
    areas = anchors[:, 0] * anchors[:, 1]
    anchors[:, 0] = np.sqrt(areas * np.repeat(_RATIOS, len(_SCALES)))
    anchors[:, 1] = anchors[:, 0] / np.repeat(_RATIOS, len(_SCALES))
    return anchors.astype(np.float32)


_BW = 24576


def _gen(W, stride, size, loc, width):
    q = loc + jax.lax.broadcasted_iota(jnp.int32, (1, width), 1)
    cell = jnp.floor((q.astype(jnp.float32) + 0.5) * (1.0 / 9.0)).astype(jnp.int32)
    a = q - 9 * cell
    wi = jnp.bitwise_and(cell, W - 1)
    hi = jax.lax.shift_right_logical(cell, int(np.log2(W)))
    x = (wi.astype(jnp.float32) + 0.5) * float(stride)
    y = (hi.astype(jnp.float32) + 0.5) * float(stride)
    af = a.astype(jnp.float32)
    jf = jnp.floor((af + 0.5) * (1.0 / 3.0))
    u = (af - 3.0 * jf) * (1.0 / 3.0)
    v = (jf - 1.0) * 0.5
    w = jnp.exp2(u + v) * float(size)
    h = jnp.exp2(u - v) * float(size)
    return x, y, w, h


def _anchor_kernel(out_cols, out_cols2):
    pid = pl.program_id(0)

    def emit(pieces):
        x, y, w, h = (
            parts[0] if len(parts) == 1 else jnp.concatenate(parts, axis=1)
            for parts in zip(*pieces)
        )
        out_cols[:, :] = jnp.concatenate([x, y, w, h], axis=0)
        out_cols2[:, :] = jnp.concatenate(
            [x - 0.5 * w, y - 0.5 * h, x + 0.5 * w, y + 0.5 * h], axis=0
        )

    @pl.when(pid == 0)
    def _():
        emit([_gen(64, 8, 32, 0, _BW)])

    @pl.when(pid == 1)
    def _():
        emit([
            _gen(64, 8, 32, _BW, 36864 - _BW),
            _gen(32, 16, 64, 0, 9216),
            _gen(16, 32, 128, 0, 2304),
            _gen(8, 64, 256, 0, 768),
        ])


def kernel(feat_p3, feat_p4, feat_p5, feat_p6):
    del feat_p3, feat_p4, feat_p5, feat_p6
    cols = jax.ShapeDtypeStruct((4, _N_ROWS), jnp.float32)
    big0, big1 = pl.pallas_call(
        _anchor_kernel,
        grid=(2,),
        out_specs=(
            pl.BlockSpec((4, _BW), lambda i: (0, i)),
            pl.BlockSpec((4, _BW), lambda i: (0, i)),
        ),
        out_shape=(cols, cols),
    )()
    return big0.T, big1.T


# speedup vs baseline: 1.0551x; 1.0551x over previous
import numpy as np
import jax
import jax.numpy as jnp
from jax.experimental import pallas as pl

_RATIOS = np.array([0.5, 1.0, 2.0])
_SCALES = np.array([1.0, 2.0 ** (1.0 / 3.0), 2.0 ** (2.0 / 3.0)])
_LEVELS = [(64, 64, 8, 32), (32, 32, 16, 64), (16, 16, 32, 128), (8, 8, 64, 256)]
_N_ROWS = sum(h * w * 9 for h, w, _, _ in _LEVELS)
_ROW_OFF = [0, 36864, 46080, 48384]


def _box_sizes(box_size):
    anchors = box_size * np.tile(_SCALES, (2, len(_RATIOS))).T
    areas = anchors[:, 0] * anchors[:, 1]
    anchors[:, 0] = np.sqrt(areas * np.repeat(_RATIOS, len(_SCALES)))
    anchors[:, 1] = anchors[:, 0] / np.repeat(_RATIOS, len(_SCALES))
    return anchors.astype(np.float32)


def _anchor_kernel(out_cols, out_cols2):
    for lvl, (H, W, stride, size) in enumerate(_LEVELS):
        rows = H * W * 9
        seg = _ROW_OFF[lvl]
        q = jax.lax.broadcasted_iota(jnp.int32, (1, rows), 1)
        cell = jnp.floor((q.astype(jnp.float32) + 0.5) * (1.0 / 9.0)).astype(jnp.int32)
        a = q - 9 * cell
        wi = jnp.bitwise_and(cell, W - 1)
        hi = jax.lax.shift_right_logical(cell, int(np.log2(W)))
        x = (wi.astype(jnp.float32) + 0.5) * float(stride)
        y = (hi.astype(jnp.float32) + 0.5) * float(stride)

        af = a.astype(jnp.float32)
        jf = jnp.floor((af + 0.5) * (1.0 / 3.0))
        u = (af - 3.0 * jf) * (1.0 / 3.0)
        v = (jf - 1.0) * 0.5
        w = jnp.exp2(u + v) * float(size)
        h = jnp.exp2(u - v) * float(size)

        out_cols[:, seg:seg + rows] = jnp.concatenate([x, y, w, h], axis=0)
        out_cols2[:, seg:seg + rows] = jnp.concatenate(
            [x - 0.5 * w, y - 0.5 * h, x + 0.5 * w, y + 0.5 * h], axis=0
        )


def kernel(feat_p3, feat_p4, feat_p5, feat_p6):
    del feat_p3, feat_p4, feat_p5, feat_p6
    cols = jax.ShapeDtypeStruct((4, _N_ROWS), jnp.float32)
    big0, big1 = pl.pallas_call(
        _anchor_kernel,
        out_shape=(cols, cols),
    )()
    return big0.T, big1.T
